# baseline (speedup 1.0000x reference)
import jax
import jax.numpy as jnp
from jax import lax
from jax.experimental import pallas as pl
from jax.experimental.pallas import tpu as pltpu
from jax.experimental.pallas import tpu_sc as plsc

HIDDEN = 64
MAX_DEPTH = 8
NUM_TYPES = 16
K = 32

ROWS_PER_BLK = 128
BLK_TOK = ROWS_PER_BLK * 200

SC_CORES = 2
SC_SUBCORES = 16
SC_WORKERS = SC_CORES * SC_SUBCORES


def _sc_flatten(comb):
    n_rows, row_len = comb.shape
    n_blocks = (n_rows * row_len) // BLK_TOK
    blocks_per_worker = n_blocks // SC_WORKERS
    mesh = plsc.VectorSubcoreMesh(core_axis_name="c", subcore_axis_name="s")

    rows_per_step = 64
    tok_per_step = rows_per_step * row_len
    n_steps = n_rows // rows_per_step
    steps_per_worker = n_steps // SC_WORKERS

    @pl.kernel(
        out_type=jax.ShapeDtypeStruct((n_rows * row_len,), jnp.int32),
        mesh=mesh,
        scratch_types=[
            pltpu.VMEM((rows_per_step, row_len), jnp.int32),
            pltpu.VMEM((tok_per_step + 16,), jnp.int32),
            pltpu.SemaphoreType.DMA,
            pltpu.SemaphoreType.DMA,
        ],
    )
    def flatten_kernel(comb_hbm, out_hbm, buf2d, flat, in_sem, out_sem):
        wid = lax.axis_index("s") * SC_CORES + lax.axis_index("c")
        vw = 16
        chunks = -(-row_len // vw)

        @pl.loop(0, steps_per_worker)
        def _(h):
            g = wid * steps_per_worker + h
            r0 = g * rows_per_step
            pltpu.async_copy(comb_hbm.at[pl.ds(r0, rows_per_step), :],
                             buf2d, in_sem).wait()
            for r in range(rows_per_step):
                for j in range(chunks):
                    c = min(j * vw, row_len - vw)
                    flat[pl.ds(r * row_len + c, vw)] = buf2d[r, pl.ds(c, vw)]
            pltpu.async_copy(
                flat.at[pl.ds(0, tok_per_step)],
                out_hbm.at[pl.ds(r0 * row_len, tok_per_step)],
                out_sem,
            ).wait()

    return flatten_kernel(comb)


def _tc_body(comb_ref, dtab_ref, ttab_ref, w_ref, b_ref, out_ref):
    w = w_ref[...]
    pd = lax.dot_general(dtab_ref[...], w[:, :HIDDEN],
                         (((1,), (1,)), ((), ())),
                         preferred_element_type=jnp.float32) + b_ref[...]
    pt = lax.dot_general(ttab_ref[...], w[:, HIDDEN:],
                         (((1,), (1,)), ((), ())),
                         preferred_element_type=jnp.float32)
    ptab = jnp.concatenate(
        [pd, pt, jnp.zeros((K - MAX_DEPTH - NUM_TYPES, HIDDEN), jnp.float32)], axis=0)

    row = comb_ref[...][None, :]
    d = jnp.broadcast_to(row & (MAX_DEPTH - 1), (K, BLK_TOK))
    c = jnp.broadcast_to((row >> 3) + MAX_DEPTH, (K, BLK_TOK))
    iota = lax.broadcasted_iota(jnp.int32, (K, BLK_TOK), 0)
    two_hot_t = jnp.where((iota == d) | (iota == c), 1.0, 0.0)
    out_ref[...] = lax.dot_general(two_hot_t, ptab,
                                   (((0,), (0,)), ((), ())),
                                   preferred_element_type=jnp.float32)


def kernel(depth_indices, node_type_indices, depth_table, type_table, W, b):
    B, L = depth_indices.shape
    n_tok = B * L
    grid = n_tok // BLK_TOK

    comb = _sc_flatten((node_type_indices << 3) | depth_indices)

    out = pl.pallas_call(
        _tc_body,
        grid=(grid,),
        in_specs=[
            pl.BlockSpec((BLK_TOK,), lambda i: (i,)),
            pl.BlockSpec((MAX_DEPTH, HIDDEN), lambda i: (0, 0)),
            pl.BlockSpec((NUM_TYPES, HIDDEN), lambda i: (0, 0)),
            pl.BlockSpec((HIDDEN, 2 * HIDDEN), lambda i: (0, 0)),
            pl.BlockSpec((1, HIDDEN), lambda i: (0, 0)),
        ],
        out_specs=pl.BlockSpec((BLK_TOK, HIDDEN), lambda i: (i, 0)),
        out_shape=jax.ShapeDtypeStruct((n_tok, HIDDEN), jnp.float32),
        compiler_params=pltpu.CompilerParams(
            dimension_semantics=("arbitrary",)),
    )(comb, depth_table, type_table, W, b.reshape(1, HIDDEN))
    return out.reshape(B, L, HIDDEN)
